# baseline (speedup 1.0000x reference)
import functools

import jax
import jax.numpy as jnp
from jax import lax
from jax.experimental import pallas as pl
from jax.experimental.pallas import tpu as pltpu
from jax.experimental.pallas import tpu_sc as plsc

EMB_DIM = 64
NC, NS = 2, 16
NW = NC * NS
K = 128


def _make_sc_gather(B):
    b_per_w = B // NW
    n_chunks = b_per_w // K
    mesh = plsc.VectorSubcoreMesh(
        core_axis_name="c", subcore_axis_name="s",
        num_cores=NC, num_subcores=NS)

    @functools.partial(
        pl.kernel,
        out_type=jax.ShapeDtypeStruct((B, EMB_DIM), jnp.float32),
        mesh=mesh,
        scratch_types=[
            pltpu.VMEM((b_per_w,), jnp.int32),
            pltpu.VMEM((K, EMB_DIM), jnp.float32),
            pltpu.SemaphoreType.DMA,
        ],
        compiler_params=pltpu.CompilerParams(use_tc_tiling_on_sc=False),
    )
    def sc_gather(idx_hbm, table_hbm, out_hbm, idx_v, buf, sem):
        wid = lax.axis_index("s") * NC + lax.axis_index("c")
        base = wid * b_per_w
        pltpu.sync_copy(idx_hbm.at[pl.ds(base, b_per_w)], idx_v)

        def body(j, carry):
            pltpu.async_copy(
                table_hbm.at[idx_v.at[pl.ds(j * K, K)]], buf, sem).wait()
            pltpu.sync_copy(buf, out_hbm.at[pl.ds(base + j * K, K)])
            return carry

        lax.fori_loop(0, n_chunks, body, 0)

    return sc_gather


def kernel(x, emb_weight):
    b, f = x.shape
    idx = x.reshape(b * f).astype(jnp.int32)
    out = _make_sc_gather(b * f)(idx, emb_weight)
    return out.reshape(b, f, EMB_DIM)


# speedup vs baseline: 1.0772x; 1.0772x over previous
import functools

import jax
import jax.numpy as jnp
from jax import lax
from jax.experimental import pallas as pl
from jax.experimental.pallas import tpu as pltpu
from jax.experimental.pallas import tpu_sc as plsc

EMB_DIM = 64
NC, NS = 2, 16
NW = NC * NS
K = 128
NBUF = 4


def _make_sc_gather(B):
    b_per_w = B // NW
    n_chunks = b_per_w // K
    assert n_chunks % NBUF == 0 and n_chunks >= 2 * NBUF
    mesh = plsc.VectorSubcoreMesh(
        core_axis_name="c", subcore_axis_name="s",
        num_cores=NC, num_subcores=NS)

    @functools.partial(
        pl.kernel,
        out_type=jax.ShapeDtypeStruct((B, EMB_DIM), jnp.float32),
        mesh=mesh,
        scratch_types=[
            pltpu.VMEM((b_per_w,), jnp.int32),
            [pltpu.VMEM((K, EMB_DIM), jnp.float32) for _ in range(NBUF)],
            [pltpu.SemaphoreType.DMA for _ in range(NBUF)],
            [pltpu.SemaphoreType.DMA for _ in range(NBUF)],
        ],
        compiler_params=pltpu.CompilerParams(use_tc_tiling_on_sc=False),
    )
    def sc_gather(idx_hbm, table_hbm, out_hbm, idx_v, bufs, gsems, ssems):
        wid = lax.axis_index("s") * NC + lax.axis_index("c")
        base = wid * b_per_w
        pltpu.sync_copy(idx_hbm.at[pl.ds(base, b_per_w)], idx_v)

        def g_src(c):
            return table_hbm.at[idx_v.at[pl.ds(c * K, K)]]

        def o_dst(c):
            return out_hbm.at[pl.ds(base + c * K, K)]

        def start_gather(c, p):
            pltpu.async_copy(g_src(c), bufs[p], gsems[p])

        def wait_gather(c, p):
            pltpu.make_async_copy(g_src(c), bufs[p], gsems[p]).wait()

        def start_store(c, p):
            pltpu.async_copy(bufs[p], o_dst(c), ssems[p])

        def wait_store(c, p):
            pltpu.make_async_copy(bufs[p], o_dst(c), ssems[p]).wait()

        start_gather(0, 0)
        start_gather(1, 1)
        for j in (0, 1):
            start_gather(j + 2, j + 2)
            wait_gather(j, j)
            start_store(j, j)

        def body(jj, carry):
            for u in range(NBUF):
                j = 2 + NBUF * jj + u
                p = (2 + u) % NBUF
                q = u
                wait_store(j - 2, q)
                start_gather(j + 2, q)
                wait_gather(j, p)
                start_store(j, p)
            return carry

        lax.fori_loop(0, (n_chunks - 4) // NBUF, body, 0)

        for j in (n_chunks - 2, n_chunks - 1):
            wait_gather(j, j % NBUF)
            start_store(j, j % NBUF)
        for j in range(n_chunks - 4, n_chunks):
            wait_store(j, j % NBUF)

    return sc_gather


def kernel(x, emb_weight):
    b, f = x.shape
    idx = x.reshape(b * f).astype(jnp.int32)
    out = _make_sc_gather(b * f)(idx, emb_weight)
    return out.reshape(b, f, EMB_DIM)


# speedup vs baseline: 1.2630x; 1.1724x over previous
import functools

import jax
import jax.numpy as jnp
from jax import lax
from jax.experimental import pallas as pl
from jax.experimental.pallas import tpu as pltpu
from jax.experimental.pallas import tpu_sc as plsc

EMB_DIM = 64
NC, NS = 2, 16
NW = NC * NS
K = 128
TP = 133


def _make_sc_gather(n_b, n_f):
    B_units = n_b // K
    n_units = n_f * B_units
    u_per_w = n_units // NW
    idx_per_w = u_per_w * K
    mesh = plsc.VectorSubcoreMesh(
        core_axis_name="c", subcore_axis_name="s",
        num_cores=NC, num_subcores=NS)

    @functools.partial(
        pl.kernel,
        out_type=jax.ShapeDtypeStruct((n_f, 8, B_units, 8, K), jnp.float32),
        mesh=mesh,
        scratch_types=[
            pltpu.VMEM((idx_per_w,), jnp.int32),
            [pltpu.VMEM((K, EMB_DIM), jnp.float32) for _ in range(2)],
            [pltpu.VMEM((8, 8, TP), jnp.float32) for _ in range(2)],
            [pltpu.SemaphoreType.DMA for _ in range(2)],
            [pltpu.SemaphoreType.DMA for _ in range(2)],
        ],
        compiler_params=pltpu.CompilerParams(
            use_tc_tiling_on_sc=False, needs_layout_passes=False),
    )
    def sc_gather(idx_hbm, table_hbm, out_hbm, idx_v, rbufs, tbufs,
                  gsems, osems):
        wid = lax.axis_index("s") * NC + lax.axis_index("c")
        u_base = wid * u_per_w
        pltpu.sync_copy(idx_hbm.at[pl.ds(u_base * K, idx_per_w)], idx_v)

        iota = lax.iota(jnp.int32, 16)
        g_idx = [(jnp.int32(16 * j) + iota) >> 3 for j in range(4)]
        s_idx = [(jnp.int32(16 * j) + iota) & 7 for j in range(4)]

        def g_src(u_local):
            return table_hbm.at[idx_v.at[pl.ds(u_local * K, K)]]

        def o_dst(u_local):
            u = u_base + u_local
            f = u // B_units
            b = u % B_units
            return out_hbm.at[f, pl.ds(0, 8), b]

        def start_gather(u_local, p):
            pltpu.async_copy(g_src(u_local), rbufs[p], gsems[p])

        def wait_gather(u_local, p):
            pltpu.make_async_copy(g_src(u_local), rbufs[p], gsems[p]).wait()

        def start_out(u_local, p):
            pltpu.async_copy(tbufs[p].at[:, :, pl.ds(0, K)],
                             o_dst(u_local), osems[p])

        def wait_out(u_local, p):
            pltpu.make_async_copy(tbufs[p].at[:, :, pl.ds(0, K)],
                                  o_dst(u_local), osems[p]).wait()

        def transpose(p):
            rb, tb = rbufs[p], tbufs[p]

            def row(l, carry):
                l_vec = jnp.full((16,), 0, jnp.int32) + l
                for j in range(4):
                    data = rb[l, pl.ds(16 * j, 16)]
                    plsc.store_scatter(tb, [g_idx[j], s_idx[j], l_vec], data)
                return carry

            lax.fori_loop(0, K, row, 0)

        start_gather(0, 0)
        wait_gather(0, 0)
        start_gather(1, 1)
        transpose(0)
        start_out(0, 0)
        start_gather(2, 0)
        wait_gather(1, 1)
        transpose(1)
        start_out(1, 1)
        start_gather(3, 1)

        def body(jj, carry):
            for par in range(2):
                u = 2 + 2 * jj + par
                p = par
                wait_gather(u, p)
                wait_out(u - 2, p)
                transpose(p)
                start_out(u, p)
                start_gather(u + 2, p)
            return carry

        lax.fori_loop(0, (u_per_w - 4) // 2, body, 0)

        for u in (u_per_w - 2, u_per_w - 1):
            p = u % 2
            wait_gather(u, p)
            wait_out(u - 2, p)
            transpose(p)
            start_out(u, p)
        wait_out(u_per_w - 2, 0)
        wait_out(u_per_w - 1, 1)

    return sc_gather


def kernel(x, emb_weight):
    b, f = x.shape
    idx_f = x.T.reshape(b * f).astype(jnp.int32)
    y5 = _make_sc_gather(b, f)(idx_f, emb_weight)
    return y5.transpose(2, 4, 0, 1, 3).reshape(b, f, EMB_DIM)


# speedup vs baseline: 1.3643x; 1.0802x over previous
import functools

import jax
import jax.numpy as jnp
from jax import lax
from jax.experimental import pallas as pl
from jax.experimental.pallas import tpu as pltpu
from jax.experimental.pallas import tpu_sc as plsc

EMB_DIM = 64
NC, NS = 2, 16
NW = NC * NS
K = 128
TP = 133


def _make_sc_gather(n_b, n_f):
    B_units = n_b // K
    n_units = n_f * B_units
    u_per_w = n_units // NW
    idx_per_w = u_per_w * K
    mesh = plsc.VectorSubcoreMesh(
        core_axis_name="c", subcore_axis_name="s",
        num_cores=NC, num_subcores=NS)

    @functools.partial(
        pl.kernel,
        out_type=jax.ShapeDtypeStruct((n_f, 8, B_units, 8, K), jnp.float32),
        mesh=mesh,
        scratch_types=[
            pltpu.VMEM((idx_per_w,), jnp.int32),
            [pltpu.VMEM((K, 2 * EMB_DIM), jnp.float32) for _ in range(2)],
            [pltpu.VMEM((8, 8, TP), jnp.float32) for _ in range(2)],
            [pltpu.SemaphoreType.DMA for _ in range(2)],
            [pltpu.SemaphoreType.DMA for _ in range(2)],
        ],
        compiler_params=pltpu.CompilerParams(
            use_tc_tiling_on_sc=False, needs_layout_passes=False),
    )
    def sc_gather(idx_hbm, table_hbm, out_hbm, idx_v, rbufs, tbufs,
                  gsems, osems):
        wid = lax.axis_index("s") * NC + lax.axis_index("c")
        u_base = wid * u_per_w
        pltpu.sync_copy(idx_hbm.at[pl.ds(u_base * K, idx_per_w)], idx_v)

        iota = lax.iota(jnp.int32, 16)
        g_idx = [(jnp.int32(16 * j) + iota) >> 3 for j in range(4)]
        s_idx = [(jnp.int32(16 * j) + iota) & 7 for j in range(4)]

        def g_src(u_local):
            return table_hbm.at[idx_v.at[pl.ds(u_local * K, K)]]

        def o_dst(u_local):
            u = u_base + u_local
            f = u // B_units
            b = u % B_units
            return out_hbm.at[f, pl.ds(0, 8), b]

        def start_gather(u_local, p):
            pltpu.async_copy(g_src(u_local), rbufs[p], gsems[p])

        def wait_gather(u_local, p):
            pltpu.make_async_copy(g_src(u_local), rbufs[p], gsems[p]).wait()

        def start_out(u_local, p):
            pltpu.async_copy(tbufs[p].at[:, :, pl.ds(0, K)],
                             o_dst(u_local), osems[p])

        def wait_out(u_local, p):
            pltpu.make_async_copy(tbufs[p].at[:, :, pl.ds(0, K)],
                                  o_dst(u_local), osems[p]).wait()

        def transpose(p):
            rb, tb = rbufs[p], tbufs[p]

            def row(l, carry):
                l_vec = jnp.full((16,), 0, jnp.int32) + l
                for j in range(4):
                    data = rb[l, pl.ds(16 * j, 16)]
                    plsc.store_scatter(tb, [g_idx[j], s_idx[j], l_vec], data)
                return carry

            lax.fori_loop(0, K, row, 0)

        start_gather(0, 0)
        wait_gather(0, 0)
        start_gather(1, 1)
        transpose(0)
        start_out(0, 0)
        start_gather(2, 0)
        wait_gather(1, 1)
        transpose(1)
        start_out(1, 1)
        start_gather(3, 1)

        def body(jj, carry):
            for par in range(2):
                u = 2 + 2 * jj + par
                p = par
                wait_gather(u, p)
                wait_out(u - 2, p)
                transpose(p)
                start_out(u, p)
                start_gather(u + 2, p)
            return carry

        lax.fori_loop(0, (u_per_w - 4) // 2, body, 0)

        for u in (u_per_w - 2, u_per_w - 1):
            p = u % 2
            wait_gather(u, p)
            wait_out(u - 2, p)
            transpose(p)
            start_out(u, p)
        wait_out(u_per_w - 2, 0)
        wait_out(u_per_w - 1, 1)

    return sc_gather


def kernel(x, emb_weight):
    b, f = x.shape
    idx_f = x.T.reshape(b * f).astype(jnp.int32)
    wp = jnp.pad(emb_weight, ((0, 0), (0, EMB_DIM)))
    y5 = _make_sc_gather(b, f)(idx_f, wp)
    return y5.transpose(2, 4, 0, 1, 3).reshape(b, f, EMB_DIM)


# speedup vs baseline: 1.5342x; 1.1245x over previous
import functools

import jax
import jax.numpy as jnp
from jax import lax
from jax.experimental import pallas as pl
from jax.experimental.pallas import tpu as pltpu
from jax.experimental.pallas import tpu_sc as plsc

EMB_DIM = 64
NC, NS = 2, 16
NW = NC * NS
K = 128
TP = 133


def _make_sc_gather(n_b, n_f):
    B_units = n_b // K
    n_units = n_f * B_units
    u_per_w = n_units // NW
    idx_per_w = u_per_w * K
    mesh = plsc.VectorSubcoreMesh(
        core_axis_name="c", subcore_axis_name="s",
        num_cores=NC, num_subcores=NS)

    @functools.partial(
        pl.kernel,
        out_type=jax.ShapeDtypeStruct((n_f, 8, B_units, 8, K), jnp.float32),
        mesh=mesh,
        scratch_types=[
            pltpu.VMEM((idx_per_w,), jnp.int32),
            [pltpu.VMEM((K, 2 * EMB_DIM), jnp.float32) for _ in range(2)],
            [pltpu.VMEM((EMB_DIM, TP), jnp.float32) for _ in range(2)],
            [pltpu.SemaphoreType.DMA for _ in range(2)],
            [pltpu.SemaphoreType.DMA for _ in range(2)],
        ],
        compiler_params=pltpu.CompilerParams(
            use_tc_tiling_on_sc=False, needs_layout_passes=False),
    )
    def sc_gather(idx_hbm, table_hbm, out_hbm, idx_v, rbufs, tbufs,
                  gsems, osems):
        wid = lax.axis_index("s") * NC + lax.axis_index("c")
        u_base = wid * u_per_w
        pltpu.sync_copy(idx_hbm.at[pl.ds(u_base * K, idx_per_w)], idx_v)

        iota = lax.iota(jnp.int32, 16)
        c_idx = [jnp.int32(16 * j) + iota for j in range(4)]

        def g_src(u_local):
            return table_hbm.at[idx_v.at[pl.ds(u_local * K, K)]]

        def o_dst(u_local, g):
            u = u_base + u_local
            f = u // B_units
            b = u % B_units
            return out_hbm.at[f, g, b]

        def start_gather(u_local, p):
            pltpu.async_copy(g_src(u_local), rbufs[p], gsems[p])

        def wait_gather(u_local, p):
            pltpu.make_async_copy(g_src(u_local), rbufs[p], gsems[p]).wait()

        def start_out(u_local, p):
            for g in range(8):
                pltpu.async_copy(
                    tbufs[p].at[pl.ds(8 * g, 8), pl.ds(0, K)],
                    o_dst(u_local, g), osems[p])

        def wait_out(u_local, p):
            for g in range(8):
                pltpu.make_async_copy(
                    tbufs[p].at[pl.ds(8 * g, 8), pl.ds(0, K)],
                    o_dst(u_local, g), osems[p]).wait()

        def transpose(p):
            rb, tb = rbufs[p], tbufs[p]

            @plsc.parallel_loop(0, K, 1, unroll=4)
            def row(l):
                l_vec = jnp.full((16,), 0, jnp.int32) + l
                for j in range(4):
                    data = rb[l, pl.ds(16 * j, 16)]
                    plsc.store_scatter(tb, [c_idx[j], l_vec], data)

        start_gather(0, 0)
        wait_gather(0, 0)
        start_gather(1, 1)
        transpose(0)
        start_out(0, 0)
        start_gather(2, 0)
        wait_gather(1, 1)
        transpose(1)
        start_out(1, 1)
        start_gather(3, 1)

        def body(jj, carry):
            for par in range(2):
                u = 2 + 2 * jj + par
                p = par
                wait_gather(u, p)
                wait_out(u - 2, p)
                transpose(p)
                start_out(u, p)
                start_gather(u + 2, p)
            return carry

        lax.fori_loop(0, (u_per_w - 4) // 2, body, 0)

        for u in (u_per_w - 2, u_per_w - 1):
            p = u % 2
            wait_gather(u, p)
            wait_out(u - 2, p)
            transpose(p)
            start_out(u, p)
        wait_out(u_per_w - 2, 0)
        wait_out(u_per_w - 1, 1)

    return sc_gather


def kernel(x, emb_weight):
    b, f = x.shape
    idx_f = x.T.reshape(b * f).astype(jnp.int32)
    wp = jnp.pad(emb_weight, ((0, 0), (0, EMB_DIM)))
    y5 = _make_sc_gather(b, f)(idx_f, wp)
    return y5.transpose(2, 4, 0, 1, 3).reshape(b, f, EMB_DIM)


# speedup vs baseline: 1.6114x; 1.0503x over previous
import functools

import jax
import jax.numpy as jnp
from jax import lax
from jax.experimental import pallas as pl
from jax.experimental.pallas import tpu as pltpu
from jax.experimental.pallas import tpu_sc as plsc

EMB_DIM = 64
NC, NS = 2, 16
NW = NC * NS
K = 128
TP = 133


def _make_sc_gather(n_b, n_f):
    B_units = n_b // K
    n_units = n_f * B_units
    u_per_w = n_units // NW
    idx_per_w = u_per_w * K
    mesh = plsc.VectorSubcoreMesh(
        core_axis_name="c", subcore_axis_name="s",
        num_cores=NC, num_subcores=NS)

    @functools.partial(
        pl.kernel,
        out_type=jax.ShapeDtypeStruct((n_f, 8, B_units, 8, K), jnp.float32),
        mesh=mesh,
        scratch_types=[
            pltpu.VMEM((idx_per_w,), jnp.int32),
            [pltpu.VMEM((K, EMB_DIM), jnp.float32) for _ in range(2)],
            [pltpu.VMEM((EMB_DIM, TP), jnp.float32) for _ in range(2)],
            [pltpu.SemaphoreType.DMA for _ in range(2)],
            [pltpu.SemaphoreType.DMA for _ in range(2)],
        ],
        compiler_params=pltpu.CompilerParams(
            use_tc_tiling_on_sc=False, needs_layout_passes=False),
    )
    def sc_gather(idx_hbm, table_hbm, out_hbm, idx_v, rbufs, tbufs,
                  gsems, osems):
        wid = lax.axis_index("s") * NC + lax.axis_index("c")
        u_base = wid * u_per_w
        pltpu.sync_copy(idx_hbm.at[pl.ds(u_base * K, idx_per_w)], idx_v)

        iota = lax.iota(jnp.int32, 16)
        c_idx = [jnp.int32(16 * j) + iota for j in range(4)]

        def g_src(u_local):
            return table_hbm.at[idx_v.at[pl.ds(u_local * K, K)]]

        def o_dst(u_local, g):
            u = u_base + u_local
            f = u // B_units
            b = u % B_units
            return out_hbm.at[f, g, b]

        def start_gather(u_local, p):
            pltpu.async_copy(g_src(u_local), rbufs[p], gsems[p])

        def wait_gather(u_local, p):
            pltpu.make_async_copy(g_src(u_local), rbufs[p], gsems[p]).wait()

        def start_out(u_local, p):
            for g in range(8):
                pltpu.async_copy(
                    tbufs[p].at[pl.ds(8 * g, 8), pl.ds(0, K)],
                    o_dst(u_local, g), osems[p])

        def wait_out(u_local, p):
            for g in range(8):
                pltpu.make_async_copy(
                    tbufs[p].at[pl.ds(8 * g, 8), pl.ds(0, K)],
                    o_dst(u_local, g), osems[p]).wait()

        def transpose(p):
            rb, tb = rbufs[p], tbufs[p]

            @plsc.parallel_loop(0, K, 1, unroll=8)
            def row(l):
                l_vec = jnp.full((16,), 0, jnp.int32) + l
                for j in range(4):
                    data = rb[l, pl.ds(16 * j, 16)]
                    plsc.store_scatter(tb, [c_idx[j], l_vec], data)

        start_gather(0, 0)
        wait_gather(0, 0)
        start_gather(1, 1)
        transpose(0)
        start_out(0, 0)
        start_gather(2, 0)
        wait_gather(1, 1)
        transpose(1)
        start_out(1, 1)
        start_gather(3, 1)

        def body(jj, carry):
            for par in range(2):
                u = 2 + 2 * jj + par
                p = par
                wait_gather(u, p)
                wait_out(u - 2, p)
                transpose(p)
                start_out(u, p)
                start_gather(u + 2, p)
            return carry

        lax.fori_loop(0, (u_per_w - 4) // 2, body, 0)

        for u in (u_per_w - 2, u_per_w - 1):
            p = u % 2
            wait_gather(u, p)
            wait_out(u - 2, p)
            transpose(p)
            start_out(u, p)
        wait_out(u_per_w - 2, 0)
        wait_out(u_per_w - 1, 1)

    return sc_gather


def kernel(x, emb_weight):
    b, f = x.shape
    idx_f = (x.T * 2).reshape(b * f).astype(jnp.int32)
    wp = jnp.pad(emb_weight, ((0, 0), (0, EMB_DIM))).reshape(-1, EMB_DIM)
    y5 = _make_sc_gather(b, f)(idx_f, wp)
    return y5.transpose(2, 4, 0, 1, 3).reshape(b, f, EMB_DIM)
